# baseline (speedup 1.0000x reference)
import numpy as np
import jax
import jax.numpy as jnp
from jax.experimental import pallas as pl

EDGE_F = 256
NODE_F = 256
NUM_POS = 16
NUM_RBF = 16
TOP_K = 30
KNN_R = 256
EDGE_R = 64


def _norm3(c0, c1, c2, eps=1e-12):
    n = jnp.sqrt(c0 * c0 + c1 * c1 + c2 * c2)
    n = jnp.maximum(n, eps)
    return c0 / n, c1 / n, c2 / n


def _cross3(a, b):
    return (a[1] * b[2] - a[2] * b[1],
            a[2] * b[0] - a[0] * b[2],
            a[0] * b[1] - a[1] * b[0])


def _dot3(a, b):
    return a[0] * b[0] + a[1] * b[1] + a[2] * b[2]


def _cols(x):
    return (x[:, 0:1], x[:, 1:2], x[:, 2:3])


def _knn_kernel(xallT_ref, xrow_ref, mall_ref, mrow_ref, dn_ref, idx_ref):
    xallT = xallT_ref[0]
    xrow = xrow_ref[0]
    L = xallT.shape[1]
    R = xrow.shape[0]

    d2 = jnp.zeros((R, L), jnp.float32)
    for j in range(3):
        dj = xrow[:, j:j + 1] - xallT[j:j + 1, :]
        d2 = d2 + dj * dj
    mall = mall_ref[0]
    mrow = mrow_ref[0]
    m2 = mrow * mall
    D = m2 * jnp.sqrt(d2 + 1e-6)
    Dmax = jnp.max(D, axis=-1, keepdims=True)
    work = D + (1.0 - m2) * Dmax

    iota = jax.lax.broadcasted_iota(jnp.int32, (R, L), 1)
    vals = []
    idxs = []
    for _ in range(TOP_K):
        mv = jnp.min(work, axis=-1, keepdims=True)
        eq = work == mv
        ik = jnp.min(jnp.where(eq, iota, L), axis=-1, keepdims=True)
        vals.append(mv)
        idxs.append(ik)
        work = jnp.where(iota == ik, jnp.float32(np.inf), work)
    dn_ref[0] = jnp.concatenate(vals, axis=1)
    idx_ref[0] = jnp.concatenate(idxs, axis=1)


def _node_kernel(xn_ref, xca_ref, xc_ref, wn_ref, bn_ref, gn_ref,
                 betan_ref, v_ref, o_ref):
    xn = xn_ref[0]
    xca = xca_ref[0]
    xc = xc_ref[0]
    L = xn.shape[0]

    p0, p1, p2 = _cols(xn), _cols(xca), _cols(xc)
    d0 = _norm3(p1[0] - p0[0], p1[1] - p0[1], p1[2] - p0[2])
    d1 = _norm3(p2[0] - p1[0], p2[1] - p1[1], p2[2] - p1[2])
    zc = jnp.zeros((1, 1), jnp.float32)
    p0s = tuple(jnp.concatenate([c[1:], zc], axis=0) for c in p0)
    d2 = _norm3(p0s[0] - p2[0], p0s[1] - p2[1], p0s[2] - p2[2])
    d2m = tuple(jnp.concatenate([zc, c[:-1]], axis=0) for c in d2)

    n_a = _norm3(*_cross3(d2m, d0))
    n_b = _norm3(*_cross3(d0, d1))
    n_c = _norm3(*_cross3(d1, d2))
    n_d = _norm3(*_cross3(d2, _shift_up3(d0)))

    riota = jax.lax.broadcasted_iota(jnp.int32, (L, 1), 0)
    first = riota == 0
    last = riota == (L - 1)

    def angle(nb2, nb1, u2, invalid):
        cosd = jnp.clip(_dot3(nb2, nb1), -1.0 + 1e-7, 1.0 - 1e-7)
        sgn = jnp.sign(_dot3(u2, nb1))
        sind = sgn * jnp.sqrt(1.0 - cosd * cosd)
        cosd = jnp.where(invalid, 1.0, cosd)
        sind = jnp.where(invalid, 0.0, sind)
        return cosd, sind

    c0, s0 = angle(n_a, n_b, d2m, first)
    c1, s1 = angle(n_b, n_c, d0, last)
    c2, s2 = angle(n_c, n_d, d1, last)
    vfeat = jnp.concatenate([c0, c1, c2, s0, s1, s2], axis=-1)

    nca = _norm3(p1[0] - p0[0], p1[1] - p0[1], p1[2] - p0[2])
    cac = _norm3(p2[0] - p1[0], p2[1] - p1[1], p2[2] - p1[2])
    n1 = _norm3(*_cross3(nca, cac))
    bv = _norm3(cac[0] - nca[0], cac[1] - nca[1], cac[2] - nca[2])
    xax = _norm3(*_cross3(bv, n1))
    o_ref[0] = jnp.concatenate(list(bv) + list(n1) + list(xax), axis=-1)

    h = jnp.dot(vfeat, wn_ref[...], preferred_element_type=jnp.float32)
    h = h + bn_ref[...]
    mu = jnp.mean(h, axis=-1, keepdims=True)
    var = jnp.mean((h - mu) ** 2, axis=-1, keepdims=True)
    v_ref[0] = (h - mu) / jnp.sqrt(var + 1e-5) * gn_ref[...] + betan_ref[...]


def _shift_up3(t):
    zc = jnp.zeros((1, 1), jnp.float32)
    return tuple(jnp.concatenate([c[1:], zc], axis=0) for c in t)


def _edge_kernel(dn_ref, idx_ref, ii_ref, otab_ref, xtab_ref, oc_ref,
                 xc_ref, we_ref, be_ref, ge_ref, betae_ref, e_ref):
    idx = idx_ref[0]
    E = idx.shape[1]
    otab = otab_ref[0]
    xtab = xtab_ref[0]
    L = otab.shape[1]

    table = jnp.concatenate([otab, xtab], axis=0)
    iota = jax.lax.broadcasted_iota(jnp.int32, (L, E), 0)
    oh = (iota == idx).astype(jnp.float32)
    g = jnp.dot(table, oh, preferred_element_type=jnp.float32,
                precision=jax.lax.Precision.HIGHEST)

    onb = [g[m:m + 1, :] for m in range(9)]
    xnb = [g[9 + j:10 + j, :] for j in range(3)]

    oc = oc_ref[0]
    xcc = xc_ref[0]
    dxn = [xnb[j] - xcc[j:j + 1, :] for j in range(3)]

    def rb(x):
        return x.astype(jnp.bfloat16).astype(jnp.float32)

    ocb = rb(oc)
    onbb = [rb(m) for m in onb]
    dxnb = [rb(dx) for dx in dxn]
    du = []
    for p in range(3):
        du.append(ocb[3 * p + 0:3 * p + 1, :] * dxnb[0]
                  + ocb[3 * p + 1:3 * p + 2, :] * dxnb[1]
                  + ocb[3 * p + 2:3 * p + 3, :] * dxnb[2])
    dun = jnp.maximum(jnp.sqrt(du[0] ** 2 + du[1] ** 2 + du[2] ** 2), 1e-12)
    du = [d / dun for d in du]

    r = {}
    for p in range(3):
        for q in range(3):
            r[(p, q)] = (ocb[0 + p:1 + p, :] * onbb[0 + q]
                         + ocb[3 + p:4 + p, :] * onbb[3 + q]
                         + ocb[6 + p:7 + p, :] * onbb[6 + q])
    r00, r11, r22 = r[(0, 0)], r[(1, 1)], r[(2, 2)]
    mx = 0.5 * jnp.sqrt(jnp.abs(1.0 + (r00 - r11 - r22) + 1e-10))
    my = 0.5 * jnp.sqrt(jnp.abs(1.0 + (-r00 + r11 - r22) + 1e-10))
    mz = 0.5 * jnp.sqrt(jnp.abs(1.0 + (-r00 - r11 + r22) + 1e-10))
    qx = jnp.sign(r[(2, 1)] - r[(1, 2)]) * mx
    qy = jnp.sign(r[(0, 2)] - r[(2, 0)]) * my
    qz = jnp.sign(r[(1, 0)] - r[(0, 1)]) * mz
    qw = jnp.sqrt(jax.nn.relu(1.0 + r00 + r11 + r22)) / 2.0
    qn = jnp.maximum(jnp.sqrt(qx * qx + qy * qy + qz * qz + qw * qw), 1e-12)
    qx, qy, qz, qw = qx / qn, qy / qn, qz / qn, qw / qn

    dnb = dn_ref[0]
    sigma = 20.0 / NUM_RBF
    rbf = [jnp.exp(-(((dnb - (20.0 * t / (NUM_RBF - 1))) / sigma) ** 2))
           for t in range(NUM_RBF)]

    dpos = idx.astype(jnp.float32) - ii_ref[0]
    freqs = np.exp(np.arange(0, NUM_POS, 2, dtype=np.float32)
                   * -(np.log(10000.0) / NUM_POS))
    cosp = [jnp.cos(dpos * float(f)) for f in freqs]
    sinp = [jnp.sin(dpos * float(f)) for f in freqs]

    feats = cosp + sinp + rbf + du + [qx, qy, qz, qw]
    F = jnp.concatenate(feats, axis=0)

    h = jnp.dot(we_ref[...], F, preferred_element_type=jnp.float32)
    h = h + be_ref[...]
    mu = jnp.mean(h, axis=0, keepdims=True)
    var = jnp.mean((h - mu) ** 2, axis=0, keepdims=True)
    y = (h - mu) / jnp.sqrt(var + 1e-5) * ge_ref[...] + betae_ref[...]
    e_ref[0] = y.T


def kernel(X, mask, Wn, bn, gn, betan, We, be, ge, betae):
    B, L = X.shape[0], X.shape[1]
    K = TOP_K
    X_ca = X[:, :, 1, :]
    X_caT = jnp.swapaxes(X_ca, 1, 2)
    mall = mask.reshape(B, 1, L)
    mcol = mask.reshape(B, L, 1)

    dn, eidx = pl.pallas_call(
        _knn_kernel,
        grid=(B, L // KNN_R),
        in_specs=[
            pl.BlockSpec((1, 3, L), lambda b, i: (b, 0, 0)),
            pl.BlockSpec((1, KNN_R, 3), lambda b, i: (b, i, 0)),
            pl.BlockSpec((1, 1, L), lambda b, i: (b, 0, 0)),
            pl.BlockSpec((1, KNN_R, 1), lambda b, i: (b, i, 0)),
        ],
        out_specs=[
            pl.BlockSpec((1, KNN_R, K), lambda b, i: (b, i, 0)),
            pl.BlockSpec((1, KNN_R, K), lambda b, i: (b, i, 0)),
        ],
        out_shape=[
            jax.ShapeDtypeStruct((B, L, K), jnp.float32),
            jax.ShapeDtypeStruct((B, L, K), jnp.int32),
        ],
    )(X_caT, X_ca, mall, mcol)

    Xn = X[:, :, 0, :]
    Xc = X[:, :, 2, :]
    V, O = pl.pallas_call(
        _node_kernel,
        grid=(B,),
        in_specs=[
            pl.BlockSpec((1, L, 3), lambda b: (b, 0, 0)),
            pl.BlockSpec((1, L, 3), lambda b: (b, 0, 0)),
            pl.BlockSpec((1, L, 3), lambda b: (b, 0, 0)),
            pl.BlockSpec((6, NODE_F), lambda b: (0, 0)),
            pl.BlockSpec((1, NODE_F), lambda b: (0, 0)),
            pl.BlockSpec((1, NODE_F), lambda b: (0, 0)),
            pl.BlockSpec((1, NODE_F), lambda b: (0, 0)),
        ],
        out_specs=[
            pl.BlockSpec((1, L, NODE_F), lambda b: (b, 0, 0)),
            pl.BlockSpec((1, L, 9), lambda b: (b, 0, 0)),
        ],
        out_shape=[
            jax.ShapeDtypeStruct((B, L, NODE_F), jnp.float32),
            jax.ShapeDtypeStruct((B, L, 9), jnp.float32),
        ],
    )(Xn, X_ca, Xc, Wn, bn.reshape(1, NODE_F), gn.reshape(1, NODE_F),
      betan.reshape(1, NODE_F))

    EB = EDGE_R * K
    dn_f = dn.reshape(B, 1, L * K)
    idx_f = eidx.reshape(B, 1, L * K)
    ii_f = jnp.repeat(jnp.arange(L, dtype=jnp.float32), K).reshape(1, 1, L * K)
    OT = jnp.swapaxes(O, 1, 2)
    OcT = jnp.repeat(OT, K, axis=2)
    XcT = jnp.repeat(X_caT, K, axis=2)

    nfe = NUM_POS + NUM_RBF + 7
    E = pl.pallas_call(
        _edge_kernel,
        grid=(B, (L * K) // EB),
        in_specs=[
            pl.BlockSpec((1, 1, EB), lambda b, i: (b, 0, i)),
            pl.BlockSpec((1, 1, EB), lambda b, i: (b, 0, i)),
            pl.BlockSpec((1, 1, EB), lambda b, i: (0, 0, i)),
            pl.BlockSpec((1, 9, L), lambda b, i: (b, 0, 0)),
            pl.BlockSpec((1, 3, L), lambda b, i: (b, 0, 0)),
            pl.BlockSpec((1, 9, EB), lambda b, i: (b, 0, i)),
            pl.BlockSpec((1, 3, EB), lambda b, i: (b, 0, i)),
            pl.BlockSpec((EDGE_F, nfe), lambda b, i: (0, 0)),
            pl.BlockSpec((EDGE_F, 1), lambda b, i: (0, 0)),
            pl.BlockSpec((EDGE_F, 1), lambda b, i: (0, 0)),
            pl.BlockSpec((EDGE_F, 1), lambda b, i: (0, 0)),
        ],
        out_specs=pl.BlockSpec((1, EB, EDGE_F), lambda b, i: (b, i, 0)),
        out_shape=jax.ShapeDtypeStruct((B, L * K, EDGE_F), jnp.float32),
    )(dn_f, idx_f, ii_f, OT, X_caT, OcT, XcT, We.T, be.reshape(EDGE_F, 1),
      ge.reshape(EDGE_F, 1), betae.reshape(EDGE_F, 1))

    return V, E.reshape(B, L, K, EDGE_F), eidx


# speedup vs baseline: 1.0275x; 1.0275x over previous
import numpy as np
import jax
import jax.numpy as jnp
from jax.experimental import pallas as pl

EDGE_F = 256
NODE_F = 256
NUM_POS = 16
NUM_RBF = 16
TOP_K = 30
KNN_R = 256
EDGE_R = 64


def _norm3(c0, c1, c2, eps=1e-12):
    n = jnp.sqrt(c0 * c0 + c1 * c1 + c2 * c2)
    n = jnp.maximum(n, eps)
    return c0 / n, c1 / n, c2 / n


def _cross3(a, b):
    return (a[1] * b[2] - a[2] * b[1],
            a[2] * b[0] - a[0] * b[2],
            a[0] * b[1] - a[1] * b[0])


def _dot3(a, b):
    return a[0] * b[0] + a[1] * b[1] + a[2] * b[2]


def _cols(x):
    return (x[:, 0:1], x[:, 1:2], x[:, 2:3])


def _knn_kernel(xallT_ref, xrow_ref, mall_ref, mrow_ref, dn_ref, idx_ref):
    xallT = xallT_ref[0]
    xrow = xrow_ref[0]
    L = xallT.shape[1]
    R = xrow.shape[0]

    d2 = jnp.zeros((R, L), jnp.float32)
    for j in range(3):
        dj = xrow[:, j:j + 1] - xallT[j:j + 1, :]
        d2 = d2 + dj * dj
    mall = mall_ref[0]
    mrow = mrow_ref[0]
    m2 = mrow * mall
    D = m2 * jnp.sqrt(d2 + 1e-6)
    Dmax = jnp.max(D, axis=-1, keepdims=True)
    work = D + (1.0 - m2) * Dmax

    iota = jax.lax.broadcasted_iota(jnp.int32, (R, L), 1)
    vals = []
    idxs = []
    for _ in range(TOP_K):
        mv = jnp.min(work, axis=-1, keepdims=True)
        eq = work == mv
        ik = jnp.min(jnp.where(eq, iota, L), axis=-1, keepdims=True)
        vals.append(mv)
        idxs.append(ik)
        work = jnp.where(iota == ik, jnp.float32(np.inf), work)
    dn_ref[0] = jnp.concatenate(vals, axis=1)
    idx_ref[0] = jnp.concatenate(idxs, axis=1)


def _node_kernel(xn_ref, xca_ref, xc_ref, wn_ref, bn_ref, gn_ref,
                 betan_ref, v_ref, o_ref):
    xn = xn_ref[0]
    xca = xca_ref[0]
    xc = xc_ref[0]
    L = xn.shape[0]

    p0, p1, p2 = _cols(xn), _cols(xca), _cols(xc)
    d0 = _norm3(p1[0] - p0[0], p1[1] - p0[1], p1[2] - p0[2])
    d1 = _norm3(p2[0] - p1[0], p2[1] - p1[1], p2[2] - p1[2])
    zc = jnp.zeros((1, 1), jnp.float32)
    p0s = tuple(jnp.concatenate([c[1:], zc], axis=0) for c in p0)
    d2 = _norm3(p0s[0] - p2[0], p0s[1] - p2[1], p0s[2] - p2[2])
    d2m = tuple(jnp.concatenate([zc, c[:-1]], axis=0) for c in d2)

    n_a = _norm3(*_cross3(d2m, d0))
    n_b = _norm3(*_cross3(d0, d1))
    n_c = _norm3(*_cross3(d1, d2))
    n_d = _norm3(*_cross3(d2, _shift_up3(d0)))

    riota = jax.lax.broadcasted_iota(jnp.int32, (L, 1), 0)
    first = riota == 0
    last = riota == (L - 1)

    def angle(nb2, nb1, u2, invalid):
        cosd = jnp.clip(_dot3(nb2, nb1), -1.0 + 1e-7, 1.0 - 1e-7)
        sgn = jnp.sign(_dot3(u2, nb1))
        sind = sgn * jnp.sqrt(1.0 - cosd * cosd)
        cosd = jnp.where(invalid, 1.0, cosd)
        sind = jnp.where(invalid, 0.0, sind)
        return cosd, sind

    c0, s0 = angle(n_a, n_b, d2m, first)
    c1, s1 = angle(n_b, n_c, d0, last)
    c2, s2 = angle(n_c, n_d, d1, last)
    vfeat = jnp.concatenate([c0, c1, c2, s0, s1, s2], axis=-1)

    nca = _norm3(p1[0] - p0[0], p1[1] - p0[1], p1[2] - p0[2])
    cac = _norm3(p2[0] - p1[0], p2[1] - p1[1], p2[2] - p1[2])
    n1 = _norm3(*_cross3(nca, cac))
    bv = _norm3(cac[0] - nca[0], cac[1] - nca[1], cac[2] - nca[2])
    xax = _norm3(*_cross3(bv, n1))
    o_ref[0] = jnp.concatenate(list(bv) + list(n1) + list(xax), axis=-1)

    h = jnp.dot(vfeat, wn_ref[...], preferred_element_type=jnp.float32)
    h = h + bn_ref[...]
    mu = jnp.mean(h, axis=-1, keepdims=True)
    var = jnp.mean((h - mu) ** 2, axis=-1, keepdims=True)
    v_ref[0] = (h - mu) / jnp.sqrt(var + 1e-5) * gn_ref[...] + betan_ref[...]


def _shift_up3(t):
    zc = jnp.zeros((1, 1), jnp.float32)
    return tuple(jnp.concatenate([c[1:], zc], axis=0) for c in t)


def _edge_kernel(dn_ref, idx_ref, ii_ref, otab_ref, xtab_ref, oc_ref,
                 xc_ref, we_ref, be_ref, ge_ref, betae_ref, e_ref):
    idx = idx_ref[0]
    E = idx.shape[1]
    otab = otab_ref[0]
    xtab = xtab_ref[0]
    L = otab.shape[1]

    table = jnp.concatenate([otab, xtab], axis=0)
    iota = jax.lax.broadcasted_iota(jnp.int32, (L, E), 0)
    oh = (iota == idx).astype(jnp.float32)
    g = jnp.dot(table, oh, preferred_element_type=jnp.float32,
                precision=jax.lax.Precision.HIGHEST)

    onb = [g[m:m + 1, :] for m in range(9)]
    xnb = [g[9 + j:10 + j, :] for j in range(3)]

    oc = oc_ref[0]
    xcc = xc_ref[0]
    dxn = [xnb[j] - xcc[j:j + 1, :] for j in range(3)]

    def rb(x):
        return x.astype(jnp.bfloat16).astype(jnp.float32)

    ocb = rb(oc)
    onbb = [rb(m) for m in onb]
    dxnb = [rb(dx) for dx in dxn]
    du = []
    for p in range(3):
        du.append(ocb[3 * p + 0:3 * p + 1, :] * dxnb[0]
                  + ocb[3 * p + 1:3 * p + 2, :] * dxnb[1]
                  + ocb[3 * p + 2:3 * p + 3, :] * dxnb[2])
    dun = jnp.maximum(jnp.sqrt(du[0] ** 2 + du[1] ** 2 + du[2] ** 2), 1e-12)
    du = [d / dun for d in du]

    r = {}
    for p in range(3):
        for q in range(3):
            r[(p, q)] = (ocb[0 + p:1 + p, :] * onbb[0 + q]
                         + ocb[3 + p:4 + p, :] * onbb[3 + q]
                         + ocb[6 + p:7 + p, :] * onbb[6 + q])
    r00, r11, r22 = r[(0, 0)], r[(1, 1)], r[(2, 2)]
    mx = 0.5 * jnp.sqrt(jnp.abs(1.0 + (r00 - r11 - r22) + 1e-10))
    my = 0.5 * jnp.sqrt(jnp.abs(1.0 + (-r00 + r11 - r22) + 1e-10))
    mz = 0.5 * jnp.sqrt(jnp.abs(1.0 + (-r00 - r11 + r22) + 1e-10))
    qx = jnp.sign(r[(2, 1)] - r[(1, 2)]) * mx
    qy = jnp.sign(r[(0, 2)] - r[(2, 0)]) * my
    qz = jnp.sign(r[(1, 0)] - r[(0, 1)]) * mz
    qw = jnp.sqrt(jax.nn.relu(1.0 + r00 + r11 + r22)) / 2.0
    qn = jnp.maximum(jnp.sqrt(qx * qx + qy * qy + qz * qz + qw * qw), 1e-12)
    qx, qy, qz, qw = qx / qn, qy / qn, qz / qn, qw / qn

    dnb = dn_ref[0]
    sigma = 20.0 / NUM_RBF
    rbf = [jnp.exp(-(((dnb - (20.0 * t / (NUM_RBF - 1))) / sigma) ** 2))
           for t in range(NUM_RBF)]

    dpos = idx.astype(jnp.float32) - ii_ref[0]
    freqs = np.exp(np.arange(0, NUM_POS, 2, dtype=np.float32)
                   * -(np.log(10000.0) / NUM_POS))
    cosp = [jnp.cos(dpos * float(f)) for f in freqs]
    sinp = [jnp.sin(dpos * float(f)) for f in freqs]

    feats = cosp + sinp + rbf + du + [qx, qy, qz, qw]
    F = jnp.concatenate(feats, axis=0)

    h = jnp.dot(we_ref[...], F, preferred_element_type=jnp.float32)
    h = h + be_ref[...]
    mu = jnp.mean(h, axis=0, keepdims=True)
    var = jnp.mean((h - mu) ** 2, axis=0, keepdims=True)
    e_ref[0] = (h - mu) / jnp.sqrt(var + 1e-5) * ge_ref[...] + betae_ref[...]


def kernel(X, mask, Wn, bn, gn, betan, We, be, ge, betae):
    B, L = X.shape[0], X.shape[1]
    K = TOP_K
    X_ca = X[:, :, 1, :]
    X_caT = jnp.swapaxes(X_ca, 1, 2)
    mall = mask.reshape(B, 1, L)
    mcol = mask.reshape(B, L, 1)

    dn, eidx = pl.pallas_call(
        _knn_kernel,
        grid=(B, L // KNN_R),
        in_specs=[
            pl.BlockSpec((1, 3, L), lambda b, i: (b, 0, 0)),
            pl.BlockSpec((1, KNN_R, 3), lambda b, i: (b, i, 0)),
            pl.BlockSpec((1, 1, L), lambda b, i: (b, 0, 0)),
            pl.BlockSpec((1, KNN_R, 1), lambda b, i: (b, i, 0)),
        ],
        out_specs=[
            pl.BlockSpec((1, KNN_R, K), lambda b, i: (b, i, 0)),
            pl.BlockSpec((1, KNN_R, K), lambda b, i: (b, i, 0)),
        ],
        out_shape=[
            jax.ShapeDtypeStruct((B, L, K), jnp.float32),
            jax.ShapeDtypeStruct((B, L, K), jnp.int32),
        ],
    )(X_caT, X_ca, mall, mcol)

    Xn = X[:, :, 0, :]
    Xc = X[:, :, 2, :]
    V, O = pl.pallas_call(
        _node_kernel,
        grid=(B,),
        in_specs=[
            pl.BlockSpec((1, L, 3), lambda b: (b, 0, 0)),
            pl.BlockSpec((1, L, 3), lambda b: (b, 0, 0)),
            pl.BlockSpec((1, L, 3), lambda b: (b, 0, 0)),
            pl.BlockSpec((6, NODE_F), lambda b: (0, 0)),
            pl.BlockSpec((1, NODE_F), lambda b: (0, 0)),
            pl.BlockSpec((1, NODE_F), lambda b: (0, 0)),
            pl.BlockSpec((1, NODE_F), lambda b: (0, 0)),
        ],
        out_specs=[
            pl.BlockSpec((1, L, NODE_F), lambda b: (b, 0, 0)),
            pl.BlockSpec((1, L, 9), lambda b: (b, 0, 0)),
        ],
        out_shape=[
            jax.ShapeDtypeStruct((B, L, NODE_F), jnp.float32),
            jax.ShapeDtypeStruct((B, L, 9), jnp.float32),
        ],
    )(Xn, X_ca, Xc, Wn, bn.reshape(1, NODE_F), gn.reshape(1, NODE_F),
      betan.reshape(1, NODE_F))

    EB = EDGE_R * K
    dn_f = dn.reshape(B, 1, L * K)
    idx_f = eidx.reshape(B, 1, L * K)
    ii_f = jnp.repeat(jnp.arange(L, dtype=jnp.float32), K).reshape(1, 1, L * K)
    OT = jnp.swapaxes(O, 1, 2)
    OcT = jnp.repeat(OT, K, axis=2)
    XcT = jnp.repeat(X_caT, K, axis=2)

    nfe = NUM_POS + NUM_RBF + 7
    E = pl.pallas_call(
        _edge_kernel,
        grid=(B, (L * K) // EB),
        in_specs=[
            pl.BlockSpec((1, 1, EB), lambda b, i: (b, 0, i)),
            pl.BlockSpec((1, 1, EB), lambda b, i: (b, 0, i)),
            pl.BlockSpec((1, 1, EB), lambda b, i: (0, 0, i)),
            pl.BlockSpec((1, 9, L), lambda b, i: (b, 0, 0)),
            pl.BlockSpec((1, 3, L), lambda b, i: (b, 0, 0)),
            pl.BlockSpec((1, 9, EB), lambda b, i: (b, 0, i)),
            pl.BlockSpec((1, 3, EB), lambda b, i: (b, 0, i)),
            pl.BlockSpec((EDGE_F, nfe), lambda b, i: (0, 0)),
            pl.BlockSpec((EDGE_F, 1), lambda b, i: (0, 0)),
            pl.BlockSpec((EDGE_F, 1), lambda b, i: (0, 0)),
            pl.BlockSpec((EDGE_F, 1), lambda b, i: (0, 0)),
        ],
        out_specs=pl.BlockSpec((1, EDGE_F, EB), lambda b, i: (b, 0, i)),
        out_shape=jax.ShapeDtypeStruct((B, EDGE_F, L * K), jnp.float32),
    )(dn_f, idx_f, ii_f, OT, X_caT, OcT, XcT, We.T, be.reshape(EDGE_F, 1),
      ge.reshape(EDGE_F, 1), betae.reshape(EDGE_F, 1))

    return V, jnp.swapaxes(E, 1, 2).reshape(B, L, K, EDGE_F), eidx
